# baseline (speedup 1.0000x reference)
import functools

import jax
import jax.numpy as jnp
from jax.experimental import pallas as pl

N = 10000
E = 320000
D_IN = 128
HEADS = 36
D_OUT = 128

_ROW_BLK = 400


def _l1_body(x_ref, w_ref, asrc_ref, adst_ref, h_ref, as_ref, ad_ref):
    h = jnp.dot(x_ref[...], w_ref[...], preferred_element_type=jnp.float32)
    h_ref[...] = h
    as_ref[...] = jnp.dot(h, asrc_ref[...], preferred_element_type=jnp.float32)
    ad_ref[...] = jnp.dot(h, adst_ref[...], preferred_element_type=jnp.float32)


def _layer1_dense(x, W1, A_src, A_dst):
    grid = (N // _ROW_BLK,)
    return pl.pallas_call(
        _l1_body,
        grid=grid,
        in_specs=[
            pl.BlockSpec((_ROW_BLK, D_IN), lambda i: (i, 0)),
            pl.BlockSpec((D_IN, HEADS * HEADS), lambda i: (0, 0)),
            pl.BlockSpec((HEADS * HEADS, 48), lambda i: (0, 0)),
            pl.BlockSpec((HEADS * HEADS, 48), lambda i: (0, 0)),
        ],
        out_specs=[
            pl.BlockSpec((_ROW_BLK, HEADS * HEADS), lambda i: (i, 0)),
            pl.BlockSpec((_ROW_BLK, 48), lambda i: (i, 0)),
            pl.BlockSpec((_ROW_BLK, 48), lambda i: (i, 0)),
        ],
        out_shape=[
            jax.ShapeDtypeStruct((N, HEADS * HEADS), jnp.float32),
            jax.ShapeDtypeStruct((N, 48), jnp.float32),
            jax.ShapeDtypeStruct((N, 48), jnp.float32),
        ],
    )(x, W1, A_src, A_dst)


def _l2_body(acc_ref, b1_ref, w2_ref, att2_ref, h2_ref, a2_ref):
    h1 = jnp.maximum(acc_ref[...] + b1_ref[...], 0.0)
    h2 = jnp.dot(h1, w2_ref[...], preferred_element_type=jnp.float32)
    h2_ref[...] = h2
    a2_ref[...] = jnp.dot(h2, att2_ref[...], preferred_element_type=jnp.float32)


def _layer2_dense(acc1, bias1, W2, Att2):
    grid = (N // _ROW_BLK,)
    return pl.pallas_call(
        _l2_body,
        grid=grid,
        in_specs=[
            pl.BlockSpec((_ROW_BLK, HEADS * HEADS), lambda i: (i, 0)),
            pl.BlockSpec((1, HEADS * HEADS), lambda i: (0, 0)),
            pl.BlockSpec((HEADS * HEADS, D_OUT), lambda i: (0, 0)),
            pl.BlockSpec((D_OUT, 8), lambda i: (0, 0)),
        ],
        out_specs=[
            pl.BlockSpec((_ROW_BLK, D_OUT), lambda i: (i, 0)),
            pl.BlockSpec((_ROW_BLK, 8), lambda i: (i, 0)),
        ],
        out_shape=[
            jax.ShapeDtypeStruct((N, D_OUT), jnp.float32),
            jax.ShapeDtypeStruct((N, 8), jnp.float32),
        ],
    )(acc1, bias1, W2, Att2)


def _final_body(acc_ref, b2_ref, o_ref):
    o_ref[...] = jnp.maximum(acc_ref[...] + b2_ref[...], 0.0)


def _final(acc2, bias2):
    return pl.pallas_call(
        _final_body,
        grid=(N // _ROW_BLK,),
        in_specs=[
            pl.BlockSpec((_ROW_BLK, D_OUT), lambda i: (i, 0)),
            pl.BlockSpec((1, D_OUT), lambda i: (0, 0)),
        ],
        out_specs=pl.BlockSpec((_ROW_BLK, D_OUT), lambda i: (i, 0)),
        out_shape=jax.ShapeDtypeStruct((N, D_OUT), jnp.float32),
    )(acc2, bias2)


def kernel(x, edge_index, W1, att_src1, att_dst1, bias1, W2, att_src2, att_dst2, bias2):
    ei = edge_index.astype(jnp.int32)
    loop = jnp.arange(N, dtype=jnp.int32)
    src = jnp.concatenate([ei[0], loop])
    dst = jnp.concatenate([ei[1], loop])

    eye = jnp.eye(HEADS, dtype=jnp.float32)
    A_src = (att_src1[0][:, :, None] * eye[:, None, :]).reshape(HEADS * HEADS, HEADS)
    A_src = jnp.pad(A_src, ((0, 0), (0, 48 - HEADS)))
    A_dst = (att_dst1[0][:, :, None] * eye[:, None, :]).reshape(HEADS * HEADS, HEADS)
    A_dst = jnp.pad(A_dst, ((0, 0), (0, 48 - HEADS)))

    h, a_src, a_dst = _layer1_dense(x, W1, A_src, A_dst)
    a_src = a_src[:, :HEADS]
    a_dst = a_dst[:, :HEADS]

    alpha = jnp.exp(jax.nn.leaky_relu(a_src[src] + a_dst[dst], negative_slope=0.2))
    asum = jax.ops.segment_sum(alpha, dst, num_segments=N)
    w = alpha * (1.0 / (asum + 1e-16))[dst]

    hh = h.reshape(N, HEADS, HEADS)
    msg = hh[src] * w[..., None]
    acc1 = jax.ops.segment_sum(msg, dst, num_segments=N).reshape(N, HEADS * HEADS)

    Att2 = jnp.concatenate(
        [att_src2[0].T, att_dst2[0].T, jnp.zeros((D_OUT, 6), jnp.float32)], axis=1)
    h2, a2 = _layer2_dense(acc1, bias1.reshape(1, -1), W2, Att2)

    alpha2 = jnp.exp(jax.nn.leaky_relu(a2[src, 0] + a2[dst, 1], negative_slope=0.2))
    asum2 = jax.ops.segment_sum(alpha2, dst, num_segments=N)
    w2 = alpha2 * (1.0 / (asum2 + 1e-16))[dst]
    acc2 = jax.ops.segment_sum(h2[src] * w2[:, None], dst, num_segments=N)

    return _final(acc2, bias2.reshape(1, -1))


# speedup vs baseline: 2.4357x; 2.4357x over previous
import functools

import jax
import jax.numpy as jnp
from jax import lax
from jax.experimental import pallas as pl
from jax.experimental.pallas import tpu as pltpu
from jax.experimental.pallas import tpu_sc as plsc

N = 10000
E = 320000
D_IN = 128
HEADS = 36
D_OUT = 128

_ROW_BLK = 400

_B = 128
_NTILE = 16
_E_TOT = E + N
_E_PAD = 331776
_PT = _E_PAD // _NTILE
_NB = _PT // _B
_CHUNK = 128
_N_PAD = 10240
_NROW_T = _N_PAD // _NTILE


def _l1_body(x_ref, w_ref, asrc_ref, adst_ref, h_ref, as_ref, ad_ref):
    h = jnp.dot(x_ref[...], w_ref[...], preferred_element_type=jnp.float32)
    h_ref[...] = h
    as_ref[...] = jnp.dot(h, asrc_ref[...], preferred_element_type=jnp.float32)
    ad_ref[...] = jnp.dot(h, adst_ref[...], preferred_element_type=jnp.float32)


def _layer1_dense(x, W1, A_src, A_dst):
    grid = (N // _ROW_BLK,)
    return pl.pallas_call(
        _l1_body,
        grid=grid,
        in_specs=[
            pl.BlockSpec((_ROW_BLK, D_IN), lambda i: (i, 0)),
            pl.BlockSpec((D_IN, HEADS * HEADS), lambda i: (0, 0)),
            pl.BlockSpec((HEADS * HEADS, 48), lambda i: (0, 0)),
            pl.BlockSpec((HEADS * HEADS, 48), lambda i: (0, 0)),
        ],
        out_specs=[
            pl.BlockSpec((_ROW_BLK, HEADS * HEADS), lambda i: (i, 0)),
            pl.BlockSpec((_ROW_BLK, 48), lambda i: (i, 0)),
            pl.BlockSpec((_ROW_BLK, 48), lambda i: (i, 0)),
        ],
        out_shape=[
            jax.ShapeDtypeStruct((N, HEADS * HEADS), jnp.float32),
            jax.ShapeDtypeStruct((N, 48), jnp.float32),
            jax.ShapeDtypeStruct((N, 48), jnp.float32),
        ],
    )(x, W1, A_src, A_dst)


def _l2_body(acc_ref, b1_ref, w2_ref, att2_ref, h2_ref, a2_ref):
    h1 = jnp.maximum(acc_ref[...] + b1_ref[...], 0.0)
    h2 = jnp.dot(h1, w2_ref[...], preferred_element_type=jnp.float32)
    h2_ref[...] = h2
    a2_ref[...] = jnp.dot(h2, att2_ref[...], preferred_element_type=jnp.float32)


def _layer2_dense(acc1, bias1, W2, Att2):
    grid = (N // _ROW_BLK,)
    return pl.pallas_call(
        _l2_body,
        grid=grid,
        in_specs=[
            pl.BlockSpec((_ROW_BLK, HEADS * HEADS), lambda i: (i, 0)),
            pl.BlockSpec((1, HEADS * HEADS), lambda i: (0, 0)),
            pl.BlockSpec((HEADS * HEADS, D_OUT), lambda i: (0, 0)),
            pl.BlockSpec((D_OUT, 8), lambda i: (0, 0)),
        ],
        out_specs=[
            pl.BlockSpec((_ROW_BLK, D_OUT), lambda i: (i, 0)),
            pl.BlockSpec((_ROW_BLK, 8), lambda i: (i, 0)),
        ],
        out_shape=[
            jax.ShapeDtypeStruct((N, D_OUT), jnp.float32),
            jax.ShapeDtypeStruct((N, 8), jnp.float32),
        ],
    )(acc1, bias1, W2, Att2)


def _agg1_body(h11, srcp, dstp, w48, out, rows, zbuf, wv2, sidx, didx, acc, sem):
    c = lax.axis_index("c")
    s = lax.axis_index("s")
    zero16 = jnp.zeros((16,), jnp.float32)
    iota = lax.iota(jnp.int32, 16)

    def _zero_zbuf(r, carry):
        for v in range(_CHUNK // 16):
            zbuf[r, pl.ds(16 * v, 16)] = zero16
        return carry

    lax.fori_loop(0, _B, _zero_zbuf, 0)

    for t in range(6):
        j = 2 * t + c
        valid = j < 11
        hv = [(iota + 16 * v + j * _CHUNK) // 36 for v in range(_CHUNK // 16)]
        row0 = s * _NROW_T

        @pl.when(valid)
        def _zero_acc():
            for k in range(5):
                pltpu.sync_copy(
                    zbuf, acc.at[pl.ds(pl.multiple_of(row0 + k * _B, _B), _B)])

        plsc.subcore_barrier()

        @pl.when(valid)
        def _edges():
            def _batch(b, carry):
                base = s * _PT + b * _B
                pltpu.sync_copy(srcp.at[pl.ds(base, _B)], sidx)
                pltpu.sync_copy(dstp.at[pl.ds(base, _B)], didx)
                pltpu.sync_copy(w48.at[pl.ds(base, _B)], wv2)
                jbase = j * N
                for k8 in range(_B // 16):
                    sidx[pl.ds(16 * k8, 16)] = sidx[pl.ds(16 * k8, 16)] + jbase
                pltpu.async_copy(h11.at[sidx], rows, sem).wait()

                def _weight(i, cc):
                    irow = jnp.full((16,), i, jnp.int32)
                    for v in range(_CHUNK // 16):
                        wvals = plsc.load_gather(wv2, [irow, hv[v]])
                        rows[i, pl.ds(16 * v, 16)] = (
                            rows[i, pl.ds(16 * v, 16)] * wvals)
                    return cc

                lax.fori_loop(0, _B, _weight, 0)
                pltpu.sync_copy(rows, acc.at[didx], add=True)
                return carry

            lax.fori_loop(0, _NB, _batch, 0)

        plsc.subcore_barrier()

        @pl.when(valid)
        def _copy_out():
            for k in range(5):
                off = pl.multiple_of(j * _N_PAD + row0 + k * _B, _B)
                pltpu.sync_copy(
                    acc.at[pl.ds(pl.multiple_of(row0 + k * _B, _B), _B)],
                    out.at[pl.ds(off, _B)])


def _agg1(h11, srcp, dstp, w48):
    mesh = plsc.VectorSubcoreMesh(core_axis_name="c", subcore_axis_name="s")
    f = functools.partial(
        pl.kernel,
        mesh=mesh,
        compiler_params=pltpu.CompilerParams(
            use_tc_tiling_on_sc=False, needs_layout_passes=False),
        out_type=jax.ShapeDtypeStruct((11 * _N_PAD, _CHUNK), jnp.float32),
        scratch_types=[
            pltpu.VMEM((_B, _CHUNK), jnp.float32),
            pltpu.VMEM((_B, _CHUNK), jnp.float32),
            pltpu.VMEM((_B, 48), jnp.float32),
            pltpu.VMEM((_B,), jnp.int32),
            pltpu.VMEM((_B,), jnp.int32),
            pltpu.VMEM_SHARED((_N_PAD, _CHUNK), jnp.float32),
            pltpu.SemaphoreType.DMA,
        ],
    )(_agg1_body)
    return f(h11, srcp, dstp, w48)


def _final_body(acc_ref, b2_ref, o_ref):
    o_ref[...] = jnp.maximum(acc_ref[...] + b2_ref[...], 0.0)


def _final(acc2, bias2):
    return pl.pallas_call(
        _final_body,
        grid=(N // _ROW_BLK,),
        in_specs=[
            pl.BlockSpec((_ROW_BLK, D_OUT), lambda i: (i, 0)),
            pl.BlockSpec((1, D_OUT), lambda i: (0, 0)),
        ],
        out_specs=pl.BlockSpec((_ROW_BLK, D_OUT), lambda i: (i, 0)),
        out_shape=jax.ShapeDtypeStruct((N, D_OUT), jnp.float32),
    )(acc2, bias2)


def kernel(x, edge_index, W1, att_src1, att_dst1, bias1, W2, att_src2, att_dst2, bias2):
    ei = edge_index.astype(jnp.int32)
    loop = jnp.arange(N, dtype=jnp.int32)
    src = jnp.concatenate([ei[0], loop])
    dst = jnp.concatenate([ei[1], loop])

    eye = jnp.eye(HEADS, dtype=jnp.float32)
    A_src = (att_src1[0][:, :, None] * eye[:, None, :]).reshape(HEADS * HEADS, HEADS)
    A_src = jnp.pad(A_src, ((0, 0), (0, 48 - HEADS)))
    A_dst = (att_dst1[0][:, :, None] * eye[:, None, :]).reshape(HEADS * HEADS, HEADS)
    A_dst = jnp.pad(A_dst, ((0, 0), (0, 48 - HEADS)))

    h, a_src, a_dst = _layer1_dense(x, W1, A_src, A_dst)
    a_src = a_src[:, :HEADS]
    a_dst = a_dst[:, :HEADS]

    alpha = jnp.exp(jax.nn.leaky_relu(a_src[src] + a_dst[dst], negative_slope=0.2))
    asum = jax.ops.segment_sum(alpha, dst, num_segments=N)
    w = alpha * (1.0 / (asum + 1e-16))[dst]

    h_pad = jnp.pad(h, ((0, 0), (0, 11 * _CHUNK - HEADS * HEADS)))
    h11 = h_pad.reshape(N, 11, _CHUNK).transpose(1, 0, 2).reshape(11 * N, _CHUNK)
    pad = _E_PAD - _E_TOT
    srcp = jnp.concatenate([src, jnp.zeros((pad,), jnp.int32)])
    dstp = jnp.concatenate([dst, jnp.zeros((pad,), jnp.int32)])
    w48 = jnp.pad(w, ((0, pad), (0, 48 - HEADS)))
    acc11 = _agg1(h11, srcp, dstp, w48).reshape(11, _N_PAD, _CHUNK)
    acc1 = acc11[:, :N].transpose(1, 0, 2).reshape(N, 11 * _CHUNK)[:, :HEADS * HEADS]

    Att2 = jnp.concatenate(
        [att_src2[0].T, att_dst2[0].T, jnp.zeros((D_OUT, 6), jnp.float32)], axis=1)
    h2, a2 = _layer2_dense(acc1, bias1.reshape(1, -1), W2, Att2)

    alpha2 = jnp.exp(jax.nn.leaky_relu(a2[src, 0] + a2[dst, 1], negative_slope=0.2))
    asum2 = jax.ops.segment_sum(alpha2, dst, num_segments=N)
    w2 = alpha2 * (1.0 / (asum2 + 1e-16))[dst]
    acc2 = jax.ops.segment_sum(h2[src] * w2[:, None], dst, num_segments=N)

    return _final(acc2, bias2.reshape(1, -1))


# speedup vs baseline: 2.6763x; 1.0988x over previous
import functools

import jax
import jax.numpy as jnp
from jax import lax
from jax.experimental import pallas as pl
from jax.experimental.pallas import tpu as pltpu
from jax.experimental.pallas import tpu_sc as plsc

N = 10000
E = 320000
D_IN = 128
HEADS = 36
D_OUT = 128

_ROW_BLK = 400

_B = 128
_NTILE = 16
_E_TOT = E + N
_E_PAD = 331776
_PT = _E_PAD // _NTILE
_NB = _PT // _B
_CHUNK = 128
_N_PAD = 10240
_NROW_T = _N_PAD // _NTILE


def _l1_body(x_ref, w_ref, asrc_ref, adst_ref, h_ref, as_ref, ad_ref):
    h = jnp.dot(x_ref[...], w_ref[...], preferred_element_type=jnp.float32)
    h_ref[...] = h
    as_ref[...] = jnp.dot(h, asrc_ref[...], preferred_element_type=jnp.float32)
    ad_ref[...] = jnp.dot(h, adst_ref[...], preferred_element_type=jnp.float32)


def _layer1_dense(x, W1, A_src, A_dst):
    grid = (N // _ROW_BLK,)
    return pl.pallas_call(
        _l1_body,
        grid=grid,
        in_specs=[
            pl.BlockSpec((_ROW_BLK, D_IN), lambda i: (i, 0)),
            pl.BlockSpec((D_IN, HEADS * HEADS), lambda i: (0, 0)),
            pl.BlockSpec((HEADS * HEADS, 48), lambda i: (0, 0)),
            pl.BlockSpec((HEADS * HEADS, 48), lambda i: (0, 0)),
        ],
        out_specs=[
            pl.BlockSpec((_ROW_BLK, HEADS * HEADS), lambda i: (i, 0)),
            pl.BlockSpec((_ROW_BLK, 48), lambda i: (i, 0)),
            pl.BlockSpec((_ROW_BLK, 48), lambda i: (i, 0)),
        ],
        out_shape=[
            jax.ShapeDtypeStruct((N, HEADS * HEADS), jnp.float32),
            jax.ShapeDtypeStruct((N, 48), jnp.float32),
            jax.ShapeDtypeStruct((N, 48), jnp.float32),
        ],
    )(x, W1, A_src, A_dst)


def _l2_body(acc_ref, b1_ref, w2_ref, att2_ref, h2_ref, a2_ref):
    h1 = jnp.maximum(acc_ref[...] + b1_ref[...], 0.0)
    h2 = jnp.dot(h1, w2_ref[...], preferred_element_type=jnp.float32)
    h2_ref[...] = h2
    a2_ref[...] = jnp.dot(h2, att2_ref[...], preferred_element_type=jnp.float32)


def _layer2_dense(acc1, bias1, W2, Att2):
    grid = (N // _ROW_BLK,)
    return pl.pallas_call(
        _l2_body,
        grid=grid,
        in_specs=[
            pl.BlockSpec((_ROW_BLK, HEADS * HEADS), lambda i: (i, 0)),
            pl.BlockSpec((1, HEADS * HEADS), lambda i: (0, 0)),
            pl.BlockSpec((HEADS * HEADS, D_OUT), lambda i: (0, 0)),
            pl.BlockSpec((D_OUT, 8), lambda i: (0, 0)),
        ],
        out_specs=[
            pl.BlockSpec((_ROW_BLK, D_OUT), lambda i: (i, 0)),
            pl.BlockSpec((_ROW_BLK, 8), lambda i: (i, 0)),
        ],
        out_shape=[
            jax.ShapeDtypeStruct((N, D_OUT), jnp.float32),
            jax.ShapeDtypeStruct((N, 8), jnp.float32),
        ],
    )(acc1, bias1, W2, Att2)


def _agg1_body(h11, srcp, dstp, w48, out, rows, zbuf, wv2, sidx, didx, acc, sem):
    c = lax.axis_index("c")
    s = lax.axis_index("s")
    zero16 = jnp.zeros((16,), jnp.float32)
    iota = lax.iota(jnp.int32, 16)

    def _zero_zbuf(r, carry):
        for v in range(_CHUNK // 16):
            zbuf[r, pl.ds(16 * v, 16)] = zero16
        return carry

    lax.fori_loop(0, _B, _zero_zbuf, 0)

    for t in range(6):
        j = 2 * t + c
        valid = j < 11
        hv = [(iota + 16 * v + j * _CHUNK) // 36 for v in range(_CHUNK // 16)]
        row0 = s * _NROW_T

        @pl.when(valid)
        def _zero_acc():
            for k in range(5):
                pltpu.sync_copy(
                    zbuf, acc.at[pl.ds(pl.multiple_of(row0 + k * _B, _B), _B)])

        plsc.subcore_barrier()

        @pl.when(valid)
        def _edges():
            def _batch(b, carry):
                base = s * _PT + b * _B
                pltpu.sync_copy(srcp.at[pl.ds(base, _B)], sidx)
                pltpu.sync_copy(dstp.at[pl.ds(base, _B)], didx)
                pltpu.sync_copy(w48.at[pl.ds(base, _B)], wv2)
                jbase = j * N
                for k8 in range(_B // 16):
                    sidx[pl.ds(16 * k8, 16)] = sidx[pl.ds(16 * k8, 16)] + jbase
                pltpu.async_copy(h11.at[sidx], rows, sem).wait()

                def _weight(i, cc):
                    irow = jnp.full((16,), i, jnp.int32)
                    for v in range(_CHUNK // 16):
                        wvals = plsc.load_gather(wv2, [irow, hv[v]])
                        rows[i, pl.ds(16 * v, 16)] = (
                            rows[i, pl.ds(16 * v, 16)] * wvals)
                    return cc

                lax.fori_loop(0, _B, _weight, 0)
                pltpu.sync_copy(rows, acc.at[didx], add=True)
                return carry

            lax.fori_loop(0, _NB, _batch, 0)

        plsc.subcore_barrier()

        @pl.when(valid)
        def _copy_out():
            for k in range(5):
                off = pl.multiple_of(j * _N_PAD + row0 + k * _B, _B)
                pltpu.sync_copy(
                    acc.at[pl.ds(pl.multiple_of(row0 + k * _B, _B), _B)],
                    out.at[pl.ds(off, _B)])


def _agg1(h11, srcp, dstp, w48):
    mesh = plsc.VectorSubcoreMesh(core_axis_name="c", subcore_axis_name="s")
    f = functools.partial(
        pl.kernel,
        mesh=mesh,
        compiler_params=pltpu.CompilerParams(
            use_tc_tiling_on_sc=False, needs_layout_passes=False),
        out_type=jax.ShapeDtypeStruct((11 * _N_PAD, _CHUNK), jnp.float32),
        scratch_types=[
            pltpu.VMEM((_B, _CHUNK), jnp.float32),
            pltpu.VMEM((_B, _CHUNK), jnp.float32),
            pltpu.VMEM((_B, 48), jnp.float32),
            pltpu.VMEM((_B,), jnp.int32),
            pltpu.VMEM((_B,), jnp.int32),
            pltpu.VMEM_SHARED((_N_PAD, _CHUNK), jnp.float32),
            pltpu.SemaphoreType.DMA,
        ],
    )(_agg1_body)
    return f(h11, srcp, dstp, w48)


def _agg2_body(h2, srcp, dstp, w2p, out, rows, zbuf, wv1, sidx, didx, acc, sem):
    c = lax.axis_index("c")
    s = lax.axis_index("s")
    zero16 = jnp.zeros((16,), jnp.float32)
    half = _E_PAD // 2
    pt2 = half // _NTILE
    nb2 = pt2 // _B

    def _zero_zbuf(r, carry):
        for v in range(_CHUNK // 16):
            zbuf[r, pl.ds(16 * v, 16)] = zero16
        return carry

    lax.fori_loop(0, _B, _zero_zbuf, 0)

    row0 = s * _NROW_T
    for k in range(5):
        pltpu.sync_copy(zbuf, acc.at[pl.ds(pl.multiple_of(row0 + k * _B, _B), _B)])
    plsc.subcore_barrier()

    def _batch(b, carry):
        base = c * half + s * pt2 + b * _B
        pltpu.sync_copy(srcp.at[pl.ds(base, _B)], sidx)
        pltpu.sync_copy(dstp.at[pl.ds(base, _B)], didx)
        pltpu.sync_copy(w2p.at[pl.ds(base, _B)], wv1)
        pltpu.async_copy(h2.at[sidx], rows, sem).wait()

        def _weight(i, cc):
            wvals = plsc.load_gather(wv1, [jnp.full((16,), i, jnp.int32)])
            for v in range(_CHUNK // 16):
                rows[i, pl.ds(16 * v, 16)] = (
                    rows[i, pl.ds(16 * v, 16)] * wvals)
            return cc

        lax.fori_loop(0, _B, _weight, 0)
        pltpu.sync_copy(rows, acc.at[didx], add=True)
        return carry

    lax.fori_loop(0, nb2, _batch, 0)
    plsc.subcore_barrier()

    for k in range(5):
        off = pl.multiple_of(c * _N_PAD + row0 + k * _B, _B)
        pltpu.sync_copy(
            acc.at[pl.ds(pl.multiple_of(row0 + k * _B, _B), _B)],
            out.at[pl.ds(off, _B)])


def _agg2(h2, srcp, dstp, w2p):
    mesh = plsc.VectorSubcoreMesh(core_axis_name="c", subcore_axis_name="s")
    f = functools.partial(
        pl.kernel,
        mesh=mesh,
        compiler_params=pltpu.CompilerParams(
            use_tc_tiling_on_sc=False, needs_layout_passes=False),
        out_type=jax.ShapeDtypeStruct((2 * _N_PAD, _CHUNK), jnp.float32),
        scratch_types=[
            pltpu.VMEM((_B, _CHUNK), jnp.float32),
            pltpu.VMEM((_B, _CHUNK), jnp.float32),
            pltpu.VMEM((_B,), jnp.float32),
            pltpu.VMEM((_B,), jnp.int32),
            pltpu.VMEM((_B,), jnp.int32),
            pltpu.VMEM_SHARED((_N_PAD, _CHUNK), jnp.float32),
            pltpu.SemaphoreType.DMA,
        ],
    )(_agg2_body)
    return f(h2, srcp, dstp, w2p)


def _final_body(a_ref, b_ref, b2_ref, o_ref):
    o_ref[...] = jnp.maximum(a_ref[...] + b_ref[...] + b2_ref[...], 0.0)


def _final(acc2a, acc2b, bias2):
    return pl.pallas_call(
        _final_body,
        grid=(N // _ROW_BLK,),
        in_specs=[
            pl.BlockSpec((_ROW_BLK, D_OUT), lambda i: (i, 0)),
            pl.BlockSpec((_ROW_BLK, D_OUT), lambda i: (i, 0)),
            pl.BlockSpec((1, D_OUT), lambda i: (0, 0)),
        ],
        out_specs=pl.BlockSpec((_ROW_BLK, D_OUT), lambda i: (i, 0)),
        out_shape=jax.ShapeDtypeStruct((N, D_OUT), jnp.float32),
    )(acc2a, acc2b, bias2)


def kernel(x, edge_index, W1, att_src1, att_dst1, bias1, W2, att_src2, att_dst2, bias2):
    ei = edge_index.astype(jnp.int32)
    loop = jnp.arange(N, dtype=jnp.int32)
    src = jnp.concatenate([ei[0], loop])
    dst = jnp.concatenate([ei[1], loop])

    eye = jnp.eye(HEADS, dtype=jnp.float32)
    A_src = (att_src1[0][:, :, None] * eye[:, None, :]).reshape(HEADS * HEADS, HEADS)
    A_src = jnp.pad(A_src, ((0, 0), (0, 48 - HEADS)))
    A_dst = (att_dst1[0][:, :, None] * eye[:, None, :]).reshape(HEADS * HEADS, HEADS)
    A_dst = jnp.pad(A_dst, ((0, 0), (0, 48 - HEADS)))

    h, a_src, a_dst = _layer1_dense(x, W1, A_src, A_dst)
    a_src = a_src[:, :HEADS]
    a_dst = a_dst[:, :HEADS]

    alpha = jnp.exp(jax.nn.leaky_relu(a_src[src] + a_dst[dst], negative_slope=0.2))
    asum = jax.ops.segment_sum(alpha, dst, num_segments=N)
    w = alpha * (1.0 / (asum + 1e-16))[dst]

    h_pad = jnp.pad(h, ((0, 0), (0, 11 * _CHUNK - HEADS * HEADS)))
    h11 = h_pad.reshape(N, 11, _CHUNK).transpose(1, 0, 2).reshape(11 * N, _CHUNK)
    pad = _E_PAD - _E_TOT
    srcp = jnp.concatenate([src, jnp.zeros((pad,), jnp.int32)])
    dstp = jnp.concatenate([dst, jnp.zeros((pad,), jnp.int32)])
    w48 = jnp.pad(w, ((0, pad), (0, 48 - HEADS)))
    acc11 = _agg1(h11, srcp, dstp, w48).reshape(11, _N_PAD, _CHUNK)
    acc1 = acc11[:, :N].transpose(1, 0, 2).reshape(N, 11 * _CHUNK)[:, :HEADS * HEADS]

    Att2 = jnp.concatenate(
        [att_src2[0].T, att_dst2[0].T, jnp.zeros((D_OUT, 6), jnp.float32)], axis=1)
    h2, a2 = _layer2_dense(acc1, bias1.reshape(1, -1), W2, Att2)

    alpha2 = jnp.exp(jax.nn.leaky_relu(a2[src, 0] + a2[dst, 1], negative_slope=0.2))
    asum2 = jax.ops.segment_sum(alpha2, dst, num_segments=N)
    w2 = alpha2 * (1.0 / (asum2 + 1e-16))[dst]

    w2p = jnp.concatenate([w2, jnp.zeros((pad,), jnp.float32)])
    acc2 = _agg2(h2, srcp, dstp, w2p).reshape(2, _N_PAD, _CHUNK)

    return _final(acc2[0, :N], acc2[1, :N], bias2.reshape(1, -1))
